# Initial kernel scaffold; baseline (speedup 1.0000x reference)
import jax
import jax.numpy as jnp
from jax.experimental import pallas as pl


def kernel(token_id, edge_src, node_type, emb, W_ih, W_hh, b_ih, b_hh, ln_g, ln_b, fc_W, fc_b):
    raise NotImplementedError("write your pallas kernel here")


# speedup vs baseline: 2.0121x; 2.0121x over previous
import functools

import jax
import jax.numpy as jnp
from jax import lax
from jax.experimental import pallas as pl
from jax.experimental.pallas import tpu as pltpu
from jax.experimental.pallas import tpu_sc as plsc

D = 256
SUB = 4
T = 15
C = 104
CPAD = 128
NW = 32
NB = 512


def _sc_embed_mean(emb, tok_cols, npad):
    n_per_w = npad // NW
    ch = 80
    n_ch = n_per_w // ch
    mesh = plsc.VectorSubcoreMesh(core_axis_name="c", subcore_axis_name="s")

    @functools.partial(
        pl.kernel,
        mesh=mesh,
        out_type=jax.ShapeDtypeStruct((npad, D), jnp.float32),
        scratch_types=(
            [pltpu.VMEM((ch,), jnp.int32) for _ in range(SUB)]
            + [pltpu.VMEM((ch, D), jnp.float32) for _ in range(SUB)]
            + [pltpu.VMEM((ch, D), jnp.float32), pltpu.SemaphoreType.DMA]
        ),
    )
    def k(emb_h, t0_h, t1_h, t2_h, t3_h, out_h,
          i0, i1, i2, i3, r0, r1, r2, r3, ob, sem):
        wid = lax.axis_index("s") * 2 + lax.axis_index("c")
        wbase = wid * n_per_w
        toks = (t0_h, t1_h, t2_h, t3_h)
        idxs = (i0, i1, i2, i3)
        rows = (r0, r1, r2, r3)
        for c in range(n_ch):
            base = wbase + c * ch
            for s in range(SUB):
                pltpu.sync_copy(toks[s].at[pl.ds(base, ch)], idxs[s])
            cps = [pltpu.async_copy(emb_h.at[idxs[s]], rows[s], sem)
                   for s in range(SUB)]
            for cp in cps:
                cp.wait()

            def jbody(j, _):
                for kk in range(D // 16):
                    sl = pl.ds(kk * 16, 16)
                    acc = r0[j, sl] + r1[j, sl] + r2[j, sl] + r3[j, sl]
                    ob[j, sl] = acc * 0.25
                return 0

            lax.fori_loop(0, ch, jbody, 0)
            pltpu.sync_copy(ob, out_h.at[pl.ds(base, ch)])

    return k(emb, *tok_cols)


def _sc_mailbox_gather(node_feat, eidx, rows_total):
    n_per_w = rows_total // NW
    ch = 120
    n_ch = n_per_w // ch
    mesh = plsc.VectorSubcoreMesh(core_axis_name="c", subcore_axis_name="s")

    @functools.partial(
        pl.kernel,
        mesh=mesh,
        out_type=jax.ShapeDtypeStruct((rows_total, D), jnp.float32),
        scratch_types=(
            [pltpu.VMEM((ch,), jnp.int32) for _ in range(2)]
            + [pltpu.VMEM((ch, D), jnp.float32) for _ in range(2)]
            + [pltpu.SemaphoreType.DMA, pltpu.SemaphoreType.DMA]
        ),
    )
    def k(nf_h, e_h, out_h, i0, i1, r0, r1, s0, s1):
        wid = lax.axis_index("s") * 2 + lax.axis_index("c")
        wbase = wid * n_per_w
        idxs = (i0, i1)
        rows = (r0, r1)
        sems = (s0, s1)

        def gbody(g, _):
            base = wbase + g * (2 * ch)
            cps = []
            for b in range(2):
                pltpu.sync_copy(e_h.at[pl.ds(base + b * ch, ch)], idxs[b])
                cps.append(pltpu.async_copy(nf_h.at[idxs[b]], rows[b], sems[b]))
            for b in range(2):
                cps[b].wait()
                pltpu.sync_copy(rows[b], out_h.at[pl.ds(base + b * ch, ch)])
            return 0

        lax.fori_loop(0, n_ch // 2, gbody, 0)

    return k(node_feat, eidx)


def _tc_gru_head(msg_tm, wihT, whhT, bih, bhh, lng, lnb, fcwT, fcb, npad):
    nblk = npad // NB

    def body(msg_r, wih_r, whh_r, bih_r, bhh_r, lng_r, lnb_r, fcw_r, fcb_r,
             out_r, h_r):
        t = pl.program_id(1)

        @pl.when(t == 0)
        def _():
            h_r[...] = jnp.zeros_like(h_r)

        x = msg_r[0]
        h = h_r[...]
        gi = jnp.dot(x, wih_r[...], preferred_element_type=jnp.float32) \
            + bih_r[...]
        gh = jnp.dot(h, whh_r[...], preferred_element_type=jnp.float32) \
            + bhh_r[...]
        r = jax.nn.sigmoid(gi[:, :D] + gh[:, :D])
        z = jax.nn.sigmoid(gi[:, D:2 * D] + gh[:, D:2 * D])
        n = jnp.tanh(gi[:, 2 * D:] + r * gh[:, 2 * D:])
        h_new = (1.0 - z) * n + z * h
        h_r[...] = h_new

        @pl.when(t == T - 1)
        def _():
            mu = jnp.mean(h_new, axis=1, keepdims=True)
            var = jnp.mean((h_new - mu) ** 2, axis=1, keepdims=True)
            y = (h_new - mu) * lax.rsqrt(var + 1e-5) * lng_r[...] + lnb_r[...]
            out_r[...] = jnp.dot(y, fcw_r[...],
                                 preferred_element_type=jnp.float32) + fcb_r[...]

    return pl.pallas_call(
        body,
        grid=(nblk, T),
        in_specs=[
            pl.BlockSpec((1, NB, D), lambda i, t: (t, i, 0)),
            pl.BlockSpec((D, 3 * D), lambda i, t: (0, 0)),
            pl.BlockSpec((D, 3 * D), lambda i, t: (0, 0)),
            pl.BlockSpec((1, 3 * D), lambda i, t: (0, 0)),
            pl.BlockSpec((1, 3 * D), lambda i, t: (0, 0)),
            pl.BlockSpec((1, D), lambda i, t: (0, 0)),
            pl.BlockSpec((1, D), lambda i, t: (0, 0)),
            pl.BlockSpec((D, CPAD), lambda i, t: (0, 0)),
            pl.BlockSpec((1, CPAD), lambda i, t: (0, 0)),
        ],
        out_specs=pl.BlockSpec((NB, CPAD), lambda i, t: (i, 0)),
        out_shape=jax.ShapeDtypeStruct((npad, CPAD), jnp.float32),
        scratch_shapes=[pltpu.VMEM((NB, D), jnp.float32)],
        compiler_params=pltpu.CompilerParams(
            dimension_semantics=("parallel", "arbitrary")),
    )(msg_tm, wihT, whhT, bih, bhh, lng, lnb, fcwT, fcb)


def kernel(token_id, edge_src, node_type, emb, W_ih, W_hh, b_ih, b_hh,
           ln_g, ln_b, fc_W, fc_b):
    n = token_id.shape[0]
    npad = ((n + 8 * NW - 1) // (8 * NW)) * (8 * NW)

    tok = token_id.astype(jnp.int32)
    tok_cols = [jnp.pad(tok[:, s], (0, npad - n)) for s in range(SUB)]
    node_feat = _sc_embed_mean(emb, tok_cols, npad)

    es = jnp.pad(edge_src[:, :T].astype(jnp.int32), ((0, npad - n), (0, 0)))
    eidx = es.T.reshape(-1)
    msg_flat = _sc_mailbox_gather(node_feat, eidx, T * npad)
    msg_tm = msg_flat.reshape(T, npad, D)

    fcwT = jnp.zeros((D, CPAD), jnp.float32).at[:, :C].set(fc_W.T)
    fcb = jnp.zeros((1, CPAD), jnp.float32).at[0, :C].set(fc_b)
    out = _tc_gru_head(msg_tm, W_ih.T, W_hh.T,
                       b_ih.reshape(1, -1), b_hh.reshape(1, -1),
                       ln_g.reshape(1, -1), ln_b.reshape(1, -1),
                       fcwT, fcb, npad)
    return out[:n, :C]
